# baseline (speedup 1.0000x reference)
import functools

import jax
import jax.numpy as jnp
from jax import lax
from jax.experimental import pallas as pl
from jax.experimental.pallas import tpu as pltpu
from jax.experimental.pallas import tpu_sc as plsc

_NF = 1_000_000
_B = 16384
_J = 200
_NC = 2
_NS = 16
_NW = _NC * _NS
_L = 16
_NG = _B // _L
_GPW = _NG // _NW
_IPG = _J * _L
_HI = 2176
_SI = _IPG - _HI
_WPAD = 1_000_448
_CHUNK = _WPAD // _NS
_NPIECE = 4
_PIECE = _CHUNK // _NPIECE


def _sc_body(ft_hbm, w_hbm, out_hbm, w_sp, stage, bufs_a, bufs_b, obuf, sem_a, sem_b):
    wid = lax.axis_index("s") * _NC + lax.axis_index("c")
    base = wid * _GPW

    sid = lax.axis_index("s")
    for r in range(_NPIECE):
        sl = pl.ds(sid * _CHUNK + r * _PIECE, _PIECE)
        pltpu.sync_copy(w_hbm.at[sl], stage)
        pltpu.sync_copy(stage, w_sp.at[sl])
    plsc.subcore_barrier()

    def fire(g, bufs, sems):
        ih, is_, vbuf = bufs
        sem_h, sem_s = sems
        goff = g * _IPG
        pltpu.sync_copy(ft_hbm.at[pl.ds(goff, _HI)], ih)
        pltpu.sync_copy(ft_hbm.at[pl.ds(goff + _HI, _SI)], is_)
        pltpu.async_copy(w_hbm.at[ih], vbuf.at[pl.ds(0, _HI)], sem_h)
        pltpu.async_copy(w_sp.at[is_], vbuf.at[pl.ds(_HI, _SI)], sem_s)

    def drain(bufs, sems):
        ih, is_, vbuf = bufs
        sem_h, sem_s = sems
        pltpu.make_async_copy(
            w_hbm.at[ih], vbuf.at[pl.ds(0, _HI)], sem_h
        ).wait()
        pltpu.make_async_copy(
            w_sp.at[is_], vbuf.at[pl.ds(_HI, _SI)], sem_s
        ).wait()

    def accum(bufs, gl):
        vbuf = bufs[-1]
        acc = jnp.zeros((_L,), jnp.float32)
        for j in range(_J):
            acc = acc + vbuf[pl.ds(j * _L, _L)]
        obuf[pl.ds(gl * _L, _L)] = acc

    fire(base, bufs_a, sem_a)

    def body(k, carry):
        fire(base + 2 * k + 1, bufs_b, sem_b)
        drain(bufs_a, sem_a)
        accum(bufs_a, 2 * k)
        fire(base + jnp.minimum(2 * k + 2, _GPW - 1), bufs_a, sem_a)
        drain(bufs_b, sem_b)
        accum(bufs_b, 2 * k + 1)
        return carry

    lax.fori_loop(0, _GPW // 2, body, 0)
    drain(bufs_a, sem_a)
    pltpu.sync_copy(obuf, out_hbm.at[pl.ds(base * _L, _GPW * _L)])


def _group_bufs():
    return (
        pltpu.VMEM((_HI,), jnp.int32),
        pltpu.VMEM((_SI,), jnp.int32),
        pltpu.VMEM((_IPG,), jnp.float32),
    )


@functools.partial(
    pl.kernel,
    out_type=jax.ShapeDtypeStruct((_B,), jnp.float32),
    mesh=plsc.VectorSubcoreMesh(core_axis_name="c", subcore_axis_name="s"),
    scratch_types=[
        pltpu.VMEM_SHARED((_WPAD,), jnp.float32),
        pltpu.VMEM((_PIECE,), jnp.float32),
        _group_bufs(),
        _group_bufs(),
        pltpu.VMEM((_GPW * _L,), jnp.float32),
        (pltpu.SemaphoreType.DMA, pltpu.SemaphoreType.DMA),
        (pltpu.SemaphoreType.DMA, pltpu.SemaphoreType.DMA),
    ],
)
def _sc_call(ft_hbm, w_hbm, out_hbm, w_sp, stage, bufs_a, bufs_b, obuf, sem_a, sem_b):
    _sc_body(ft_hbm, w_hbm, out_hbm, w_sp, stage, bufs_a, bufs_b, obuf, sem_a, sem_b)


@jax.jit
def kernel(feat_idx, W):
    ft4 = feat_idx.reshape(_NG, _L, _J).transpose(0, 2, 1).reshape(_NG * _IPG)
    w_ext = jnp.concatenate(
        [jnp.zeros((1,), W.dtype), W.reshape(_NF),
         jnp.zeros((_WPAD - _NF - 1,), W.dtype)]
    )
    out = _sc_call(ft4, w_ext)
    return out.reshape(_B, 1)


# speedup vs baseline: 1.2052x; 1.2052x over previous
import functools

import jax
import jax.numpy as jnp
from jax import lax
from jax.experimental import pallas as pl
from jax.experimental.pallas import tpu as pltpu
from jax.experimental.pallas import tpu_sc as plsc

_NF = 1_000_000
_B = 16384
_J = 200
_NC = 2
_NS = 16
_NW = _NC * _NS
_L = 16
_NG = _B // _L
_GPW = _NG // _NW
_IPG = _J * _L
_HI = 128
_SI = _IPG - _HI
_WPAD = 1_000_448
_CHUNK = _WPAD // _NS
_NPIECE = 4
_PIECE = _CHUNK // _NPIECE


def _sc_body(ft_hbm, w_hbm, out_hbm, w_sp, stage, bufs_a, bufs_b, obuf, sem_a, sem_b):
    wid = lax.axis_index("s") * _NC + lax.axis_index("c")
    base = wid * _GPW

    sid = lax.axis_index("s")
    for r in range(_NPIECE):
        sl = pl.ds(sid * _CHUNK + r * _PIECE, _PIECE)
        pltpu.sync_copy(w_hbm.at[sl], stage)
        pltpu.sync_copy(stage, w_sp.at[sl])
    plsc.subcore_barrier()

    def fire(g, bufs, sems):
        ih, is_, vbuf = bufs
        sem_h, sem_s = sems
        goff = g * _IPG
        pltpu.sync_copy(ft_hbm.at[pl.ds(goff, _HI)], ih)
        pltpu.sync_copy(ft_hbm.at[pl.ds(goff + _HI, _SI)], is_)
        pltpu.async_copy(w_hbm.at[ih], vbuf.at[pl.ds(0, _HI)], sem_h)
        pltpu.async_copy(w_sp.at[is_], vbuf.at[pl.ds(_HI, _SI)], sem_s)

    def drain(bufs, sems):
        ih, is_, vbuf = bufs
        sem_h, sem_s = sems
        pltpu.make_async_copy(
            w_hbm.at[ih], vbuf.at[pl.ds(0, _HI)], sem_h
        ).wait()
        pltpu.make_async_copy(
            w_sp.at[is_], vbuf.at[pl.ds(_HI, _SI)], sem_s
        ).wait()

    def accum(bufs, gl):
        vbuf = bufs[-1]
        acc = jnp.zeros((_L,), jnp.float32)
        for j in range(_J):
            acc = acc + vbuf[pl.ds(j * _L, _L)]
        obuf[pl.ds(gl * _L, _L)] = acc

    fire(base, bufs_a, sem_a)

    def body(k, carry):
        fire(base + 2 * k + 1, bufs_b, sem_b)
        drain(bufs_a, sem_a)
        accum(bufs_a, 2 * k)
        fire(base + jnp.minimum(2 * k + 2, _GPW - 1), bufs_a, sem_a)
        drain(bufs_b, sem_b)
        accum(bufs_b, 2 * k + 1)
        return carry

    lax.fori_loop(0, _GPW // 2, body, 0)
    drain(bufs_a, sem_a)
    pltpu.sync_copy(obuf, out_hbm.at[pl.ds(base * _L, _GPW * _L)])


def _group_bufs():
    return (
        pltpu.VMEM((_HI,), jnp.int32),
        pltpu.VMEM((_SI,), jnp.int32),
        pltpu.VMEM((_IPG,), jnp.float32),
    )


@functools.partial(
    pl.kernel,
    out_type=jax.ShapeDtypeStruct((_B,), jnp.float32),
    mesh=plsc.VectorSubcoreMesh(core_axis_name="c", subcore_axis_name="s"),
    scratch_types=[
        pltpu.VMEM_SHARED((_WPAD,), jnp.float32),
        pltpu.VMEM((_PIECE,), jnp.float32),
        _group_bufs(),
        _group_bufs(),
        pltpu.VMEM((_GPW * _L,), jnp.float32),
        (pltpu.SemaphoreType.DMA, pltpu.SemaphoreType.DMA),
        (pltpu.SemaphoreType.DMA, pltpu.SemaphoreType.DMA),
    ],
)
def _sc_call(ft_hbm, w_hbm, out_hbm, w_sp, stage, bufs_a, bufs_b, obuf, sem_a, sem_b):
    _sc_body(ft_hbm, w_hbm, out_hbm, w_sp, stage, bufs_a, bufs_b, obuf, sem_a, sem_b)


@jax.jit
def kernel(feat_idx, W):
    ft4 = feat_idx.reshape(_NG, _L, _J).transpose(0, 2, 1).reshape(_NG * _IPG)
    w_ext = jnp.concatenate(
        [jnp.zeros((1,), W.dtype), W.reshape(_NF),
         jnp.zeros((_WPAD - _NF - 1,), W.dtype)]
    )
    out = _sc_call(ft4, w_ext)
    return out.reshape(_B, 1)


# speedup vs baseline: 1.3647x; 1.1323x over previous
import functools

import jax
import jax.numpy as jnp
from jax import lax
from jax.experimental import pallas as pl
from jax.experimental.pallas import tpu as pltpu
from jax.experimental.pallas import tpu_sc as plsc

_NF = 1_000_000
_B = 16384
_J = 200
_NC = 2
_NS = 16
_NW = _NC * _NS
_L = 16
_NG = _B // _L
_GPW = _NG // _NW
_IPG = _J * _L
_HEAD = 8
_WSP = _NF + _HEAD
_PIECE = 15_632
_NP = 63
_TAIL = _NF - _NP * _PIECE
_PPT = 4


def _sc_body(ft_hbm, w_hbm, out_hbm, w_sp, stage, zbuf, bufs_a, bufs_b,
             obuf, sem_a, sem_b):
    cid = lax.axis_index("c")
    sid = lax.axis_index("s")
    wid = sid * _NC + cid
    base = wid * _GPW

    @pl.when(sid == 0)
    def _zero_head():
        zbuf[pl.ds(0, _L)] = jnp.zeros((_L,), jnp.float32)
        pltpu.sync_copy(zbuf.at[pl.ds(0, _HEAD)], w_sp.at[pl.ds(0, _HEAD)])

    for p in range(_PPT):
        k = sid * _PPT + p

        @pl.when(k < _NP)
        def _piece():
            src = pl.ds(k * _PIECE, _PIECE)
            dst = pl.ds(k * _PIECE + _HEAD, _PIECE)
            pltpu.sync_copy(w_hbm.at[src], stage)
            pltpu.sync_copy(stage, w_sp.at[dst])

    @pl.when(sid == 0)
    def _tail_piece():
        src = pl.ds(_NP * _PIECE, _TAIL)
        dst = pl.ds(_NP * _PIECE + _HEAD, _TAIL)
        pltpu.sync_copy(w_hbm.at[src], stage.at[pl.ds(0, _TAIL)])
        pltpu.sync_copy(stage.at[pl.ds(0, _TAIL)], w_sp.at[dst])

    plsc.subcore_barrier()

    def fire(g, bufs, sem):
        ibuf, vbuf = bufs
        pltpu.sync_copy(ft_hbm.at[pl.ds(g * _IPG, _IPG)], ibuf)
        pltpu.async_copy(w_sp.at[ibuf], vbuf, sem)

    def drain(bufs, sem):
        ibuf, vbuf = bufs
        pltpu.make_async_copy(w_sp.at[ibuf], vbuf, sem).wait()

    def accum(bufs, gl):
        _, vbuf = bufs
        acc = jnp.zeros((_L,), jnp.float32)
        for j in range(_J):
            acc = acc + vbuf[pl.ds(j * _L, _L)]
        obuf[pl.ds(gl * _L, _L)] = acc

    fire(base, bufs_a, sem_a)

    def body(k, carry):
        fire(base + 2 * k + 1, bufs_b, sem_b)
        drain(bufs_a, sem_a)
        accum(bufs_a, 2 * k)
        fire(base + jnp.minimum(2 * k + 2, _GPW - 1), bufs_a, sem_a)
        drain(bufs_b, sem_b)
        accum(bufs_b, 2 * k + 1)
        return carry

    lax.fori_loop(0, _GPW // 2, body, 0)
    drain(bufs_a, sem_a)
    pltpu.sync_copy(obuf, out_hbm.at[pl.ds(base * _L, _GPW * _L)])


def _group_bufs():
    return (
        pltpu.VMEM((_IPG,), jnp.int32),
        pltpu.VMEM((_IPG,), jnp.float32),
    )


@functools.partial(
    pl.kernel,
    out_type=jax.ShapeDtypeStruct((_B,), jnp.float32),
    mesh=plsc.VectorSubcoreMesh(core_axis_name="c", subcore_axis_name="s"),
    scratch_types=[
        pltpu.VMEM_SHARED((_WSP,), jnp.float32),
        pltpu.VMEM((_PIECE,), jnp.float32),
        pltpu.VMEM((_L,), jnp.float32),
        _group_bufs(),
        _group_bufs(),
        pltpu.VMEM((_GPW * _L,), jnp.float32),
        pltpu.SemaphoreType.DMA,
        pltpu.SemaphoreType.DMA,
    ],
)
def _sc_call(ft_hbm, w_hbm, out_hbm, w_sp, stage, zbuf, bufs_a, bufs_b,
             obuf, sem_a, sem_b):
    _sc_body(ft_hbm, w_hbm, out_hbm, w_sp, stage, zbuf, bufs_a, bufs_b,
             obuf, sem_a, sem_b)


@jax.jit
def kernel(feat_idx, W):
    ft4 = (
        (feat_idx.reshape(_NG, _L, _J) + 7)
        .transpose(0, 2, 1)
        .reshape(_NG * _IPG)
    )
    out = _sc_call(ft4, W.reshape(_NF))
    return out.reshape(_B, 1)


# speedup vs baseline: 1.4013x; 1.0269x over previous
import functools

import jax
import jax.numpy as jnp
from jax import lax
from jax.experimental import pallas as pl
from jax.experimental.pallas import tpu as pltpu
from jax.experimental.pallas import tpu_sc as plsc

_NF = 1_000_000
_B = 16384
_J = 200
_NC = 2
_NS = 16
_NW = _NC * _NS
_L = 16
_NG = _B // _L
_GPW = _NG // _NW
_IPG = _J * _L
_HEAD = 8
_WSP = _NF + _HEAD
_PIECE = 15_632
_NP = 63
_TAIL = _NF - _NP * _PIECE
_PPT = 4


def _sc_body(ft_hbm, w_hbm, out_hbm, w_sp, stage, zbuf,
             bufs_a, bufs_b, obuf, sem_a, sem_b):
    cid = lax.axis_index("c")
    sid = lax.axis_index("s")
    wid = sid * _NC + cid
    base = wid * _GPW

    @pl.when(sid == 0)
    def _zero_head():
        zbuf[pl.ds(0, _L)] = jnp.zeros((_L,), jnp.float32)
        pltpu.sync_copy(zbuf.at[pl.ds(0, _HEAD)], w_sp.at[pl.ds(0, _HEAD)])

    for p in range(_PPT):
        k = sid * _PPT + p

        @pl.when(k < _NP)
        def _piece():
            src = pl.ds(k * _PIECE, _PIECE)
            dst = pl.ds(k * _PIECE + _HEAD, _PIECE)
            pltpu.sync_copy(w_hbm.at[src], stage)
            pltpu.sync_copy(stage, w_sp.at[dst])

    @pl.when(sid == 0)
    def _tail_piece():
        src = pl.ds(_NP * _PIECE, _TAIL)
        dst = pl.ds(_NP * _PIECE + _HEAD, _TAIL)
        pltpu.sync_copy(w_hbm.at[src], stage.at[pl.ds(0, _TAIL)])
        pltpu.sync_copy(stage.at[pl.ds(0, _TAIL)], w_sp.at[dst])

    plsc.subcore_barrier()

    def fire(g, bufs, sem):
        ibuf, vbuf = bufs
        pltpu.sync_copy(ft_hbm.at[pl.ds(g * _IPG, _IPG)], ibuf)
        for m in range(_IPG // _L):
            sl = pl.ds(m * _L, _L)
            ibuf[sl] = ibuf[sl] + 7
        pltpu.async_copy(w_sp.at[ibuf], vbuf, sem)

    def drain(bufs, sem):
        ibuf, vbuf = bufs
        pltpu.make_async_copy(w_sp.at[ibuf], vbuf, sem).wait()

    def accum(bufs, gl):
        vbuf = bufs[-1]
        acc = jnp.zeros((_L,), jnp.float32)
        for j in range(_J):
            acc = acc + vbuf[pl.ds(j * _L, _L)]
        obuf[pl.ds(gl * _L, _L)] = acc

    fire(base, bufs_a, sem_a)

    def body(k, carry):
        fire(base + 2 * k + 1, bufs_b, sem_b)
        drain(bufs_a, sem_a)
        accum(bufs_a, 2 * k)
        fire(base + jnp.minimum(2 * k + 2, _GPW - 1), bufs_a, sem_a)
        drain(bufs_b, sem_b)
        accum(bufs_b, 2 * k + 1)
        return carry

    lax.fori_loop(0, _GPW // 2, body, 0)
    drain(bufs_a, sem_a)
    pltpu.sync_copy(obuf, out_hbm.at[pl.ds(base * _L, _GPW * _L)])


def _group_bufs():
    return (
        pltpu.VMEM((_IPG,), jnp.int32),
        pltpu.VMEM((_IPG,), jnp.float32),
    )


@functools.partial(
    pl.kernel,
    out_type=jax.ShapeDtypeStruct((_B,), jnp.float32),
    mesh=plsc.VectorSubcoreMesh(core_axis_name="c", subcore_axis_name="s"),
    scratch_types=[
        pltpu.VMEM_SHARED((_WSP,), jnp.float32),
        pltpu.VMEM((_PIECE,), jnp.float32),
        pltpu.VMEM((_L,), jnp.float32),
        _group_bufs(),
        _group_bufs(),
        pltpu.VMEM((_GPW * _L,), jnp.float32),
        pltpu.SemaphoreType.DMA,
        pltpu.SemaphoreType.DMA,
    ],
)
def _sc_call(ft_hbm, w_hbm, out_hbm, w_sp, stage, zbuf,
             bufs_a, bufs_b, obuf, sem_a, sem_b):
    _sc_body(ft_hbm, w_hbm, out_hbm, w_sp, stage, zbuf,
             bufs_a, bufs_b, obuf, sem_a, sem_b)


@jax.jit
def kernel(feat_idx, W):
    ft4 = feat_idx.reshape(_NG, _L, _J).transpose(0, 2, 1).reshape(_NG * _IPG)
    out = _sc_call(ft4, W.reshape(_NF))
    return out.reshape(_B, 1)
